# baseline (speedup 1.0000x reference)
import functools

import jax
import jax.numpy as jnp
from jax import lax
from jax.experimental import pallas as pl
from jax.experimental.pallas import tpu as pltpu
from jax.experimental.pallas import tpu_sc as plsc

B = 16384
EMBED = 32
N_GENRES = 8
NC = 2
NS = 16
NW = NC * NS
BPW = B // NW
IDX_CHUNK = 128
T_CHUNKS = BPW // IDX_CHUNK
G_CHUNKS = BPW * N_GENRES // IDX_CHUNK
G_BATCH = IDX_CHUNK // N_GENRES


def _body(title_tab, genre_tab, tidx_hbm, gidx_hbm, out_hbm,
          tidx_v, gidx_v, trows_v, grows0_v, grows1_v, outbuf_v,
          tsem, gsem0, gsem1):
    wid = lax.axis_index("s") * NC + lax.axis_index("c")
    base = wid * BPW

    for j in range(T_CHUNKS):
        pltpu.sync_copy(tidx_hbm.at[pl.ds(base + j * IDX_CHUNK, IDX_CHUNK)],
                        tidx_v.at[j])
    pltpu.sync_copy(gidx_hbm.at[pl.ds(wid * G_CHUNKS, G_CHUNKS)], gidx_v)

    tcopies = []
    for j in range(T_CHUNKS):
        tcopies.append(pltpu.async_copy(
            title_tab.at[tidx_v.at[j]],
            trows_v.at[pl.ds(j * IDX_CHUNK, IDX_CHUNK)], tsem))

    bufs = (grows0_v, grows1_v)
    sems = (gsem0, gsem1)
    copies = [None, None]
    copies[0] = pltpu.async_copy(genre_tab.at[gidx_v.at[0]], bufs[0], sems[0])
    for c in range(G_CHUNKS):
        if c + 1 < G_CHUNKS:
            nb = (c + 1) % 2
            copies[nb] = pltpu.async_copy(
                genre_tab.at[gidx_v.at[c + 1]], bufs[nb], sems[nb])
        cb = c % 2
        copies[cb].wait()
        grows_v = bufs[cb]

        def reduce_body(b, _, c=c, grows_v=grows_v):
            for k in range(EMBED // 16):
                acc = grows_v[b * N_GENRES, pl.ds(k * 16, 16)]
                for g in range(1, N_GENRES):
                    acc = acc + grows_v[b * N_GENRES + g, pl.ds(k * 16, 16)]
                outbuf_v[c * G_BATCH + b, pl.ds(EMBED + k * 16, 16)] = acc * 0.125
            return _
        lax.fori_loop(0, G_BATCH, reduce_body, None)

    for cp in tcopies:
        cp.wait()

    def merge_body(b, _):
        for k in range(EMBED // 16):
            outbuf_v[b, pl.ds(k * 16, 16)] = trows_v[b, pl.ds(k * 16, 16)]
        return _
    lax.fori_loop(0, BPW, merge_body, None)

    pltpu.sync_copy(outbuf_v, out_hbm.at[pl.ds(base, BPW)])


@jax.jit
def _run(title_table, genre_table, tidx, gidx):
    mesh = plsc.VectorSubcoreMesh(core_axis_name="c", subcore_axis_name="s",
                                  num_cores=NC, num_subcores=NS)
    return pl.kernel(
        _body,
        out_type=jax.ShapeDtypeStruct((B, 2 * EMBED), jnp.float32),
        mesh=mesh,
        scratch_types=[
            pltpu.VMEM((T_CHUNKS, IDX_CHUNK), jnp.int32),
            pltpu.VMEM((G_CHUNKS, IDX_CHUNK), jnp.int32),
            pltpu.VMEM((BPW, EMBED), jnp.float32),
            pltpu.VMEM((IDX_CHUNK, EMBED), jnp.float32),
            pltpu.VMEM((IDX_CHUNK, EMBED), jnp.float32),
            pltpu.VMEM((BPW, 2 * EMBED), jnp.float32),
            pltpu.SemaphoreType.DMA,
            pltpu.SemaphoreType.DMA,
            pltpu.SemaphoreType.DMA,
        ],
        compiler_params=pltpu.CompilerParams(use_tc_tiling_on_sc=False),
    )(title_table, genre_table, tidx, gidx)


def kernel(title_table, genre_table, movie_title, movie_genres):
    tidx = movie_title.astype(jnp.int32)
    gidx = movie_genres.astype(jnp.int32).reshape(NW * G_CHUNKS, IDX_CHUNK)
    return _run(title_table, genre_table, tidx, gidx)


# speedup vs baseline: 1.0049x; 1.0049x over previous
import jax
import jax.numpy as jnp
from jax import lax
from jax.experimental import pallas as pl
from jax.experimental.pallas import tpu as pltpu
from jax.experimental.pallas import tpu_sc as plsc

B = 16384
EMBED = 32
N_GENRES = 8
NC = 2
NS = 16
NW = NC * NS
BPW = B // NW
IDX_CHUNK = 128
T_CHUNKS = BPW // IDX_CHUNK
G_CHUNKS = BPW * N_GENRES // IDX_CHUNK
G_BATCH = IDX_CHUNK // N_GENRES


def _body(title_tab, genre_tab, tidx_hbm, gidx_hbm, out_hbm,
          tidx_v, gidx_v, trows_v, grows0_v, grows1_v, outbuf_v,
          tsem, gsem0, gsem1):
    wid = lax.axis_index("s") * NC + lax.axis_index("c")
    base = wid * BPW

    for j in range(T_CHUNKS):
        pltpu.sync_copy(tidx_hbm.at[pl.ds(base + j * IDX_CHUNK, IDX_CHUNK)],
                        tidx_v.at[j])
    pltpu.sync_copy(gidx_hbm.at[pl.ds(wid * G_CHUNKS, G_CHUNKS)], gidx_v)

    tcopies = []
    for j in range(T_CHUNKS):
        tcopies.append(pltpu.async_copy(
            title_tab.at[tidx_v.at[j]],
            trows_v.at[pl.ds(j * IDX_CHUNK, IDX_CHUNK)], tsem))

    bufs = (grows0_v, grows1_v)
    sems = (gsem0, gsem1)
    pltpu.async_copy(genre_tab.at[gidx_v.at[0]], bufs[0], sems[0])
    pltpu.async_copy(genre_tab.at[gidx_v.at[1]], bufs[1], sems[1])

    def reduce_chunk(c, grows_v):
        for b in range(G_BATCH):
            for k in range(EMBED // 16):
                acc = grows_v[b * N_GENRES, pl.ds(k * 16, 16)]
                for g in range(1, N_GENRES):
                    acc = acc + grows_v[b * N_GENRES + g, pl.ds(k * 16, 16)]
                outbuf_v[c * G_BATCH + b,
                         pl.ds(EMBED + k * 16, 16)] = acc * 0.125

    def chunk_body(c2, _):
        c = c2 * 2
        for half in range(2):
            pltpu.make_async_copy(
                genre_tab.at[gidx_v.at[0]], bufs[half], sems[half]).wait()
            reduce_chunk(c + half, bufs[half])

            @pl.when(c2 + 1 < G_CHUNKS // 2)
            def _prefetch(half=half, c=c):
                pltpu.async_copy(genre_tab.at[gidx_v.at[c + 2 + half]],
                                 bufs[half], sems[half])
        return _
    lax.fori_loop(0, G_CHUNKS // 2, chunk_body, None)

    for cp in tcopies:
        cp.wait()

    def merge_body(m, _):
        for b in range(16):
            for k in range(EMBED // 16):
                outbuf_v[m * 16 + b, pl.ds(k * 16, 16)] = (
                    trows_v[m * 16 + b, pl.ds(k * 16, 16)])
        return _
    lax.fori_loop(0, BPW // 16, merge_body, None)

    pltpu.sync_copy(outbuf_v, out_hbm.at[pl.ds(base, BPW)])


@jax.jit
def _run(title_table, genre_table, tidx, gidx):
    mesh = plsc.VectorSubcoreMesh(core_axis_name="c", subcore_axis_name="s",
                                  num_cores=NC, num_subcores=NS)
    return pl.kernel(
        _body,
        out_type=jax.ShapeDtypeStruct((B, 2 * EMBED), jnp.float32),
        mesh=mesh,
        scratch_types=[
            pltpu.VMEM((T_CHUNKS, IDX_CHUNK), jnp.int32),
            pltpu.VMEM((G_CHUNKS, IDX_CHUNK), jnp.int32),
            pltpu.VMEM((BPW, EMBED), jnp.float32),
            pltpu.VMEM((IDX_CHUNK, EMBED), jnp.float32),
            pltpu.VMEM((IDX_CHUNK, EMBED), jnp.float32),
            pltpu.VMEM((BPW, 2 * EMBED), jnp.float32),
            pltpu.SemaphoreType.DMA,
            pltpu.SemaphoreType.DMA,
            pltpu.SemaphoreType.DMA,
        ],
        compiler_params=pltpu.CompilerParams(use_tc_tiling_on_sc=False),
    )(title_table, genre_table, tidx, gidx)


def kernel(title_table, genre_table, movie_title, movie_genres):
    tidx = movie_title.astype(jnp.int32)
    gidx = movie_genres.astype(jnp.int32).reshape(NW * G_CHUNKS, IDX_CHUNK)
    return _run(title_table, genre_table, tidx, gidx)


# speedup vs baseline: 3.1536x; 3.1381x over previous
import jax
import jax.numpy as jnp
from jax import lax
from jax.experimental import pallas as pl
from jax.experimental.pallas import tpu as pltpu
from jax.experimental.pallas import tpu_sc as plsc

B = 16384
EMBED = 32
N_GENRES = 8
NC = 2
NS = 16
NW = NC * NS
BPW = B // NW
IDX_CHUNK = 128
T_CHUNKS = BPW // IDX_CHUNK
GENRE_ROWS = 21


def _body(title_tab, genre_tab, tidx_hbm, gidx_hbm, out_hbm,
          tidx_v, gidx_v, gtab_v, trows_v, outbuf_v, tsem):
    wid = lax.axis_index("s") * NC + lax.axis_index("c")
    base = wid * BPW

    for j in range(T_CHUNKS):
        pltpu.sync_copy(tidx_hbm.at[pl.ds(base + j * IDX_CHUNK, IDX_CHUNK)],
                        tidx_v.at[j])
    pltpu.sync_copy(gidx_hbm.at[pl.ds(base, BPW)], gidx_v)
    pltpu.sync_copy(genre_tab, gtab_v)

    tcopies = []
    for j in range(T_CHUNKS):
        tcopies.append(pltpu.async_copy(
            title_tab.at[tidx_v.at[j]],
            trows_v.at[pl.ds(j * IDX_CHUNK, IDX_CHUNK)], tsem))

    lanes = jnp.arange(16, dtype=jnp.int32)
    col0 = lanes % jnp.int32(16)
    cols = [col0, col0 + jnp.int32(16)]

    def genre_body(m, _):
        row0 = m * 16
        for b in range(16):
            row = row0 + b
            accs = [None, None]
            for g in range(N_GENRES):
                gid = plsc.load_gather(
                    gidx_v, [jnp.broadcast_to(row, (16,)).astype(jnp.int32),
                             jnp.broadcast_to(jnp.int32(g), (16,))])
                for k in range(EMBED // 16):
                    val = plsc.load_gather(gtab_v, [gid, cols[k]])
                    accs[k] = val if accs[k] is None else accs[k] + val
            for k in range(EMBED // 16):
                outbuf_v[row, pl.ds(EMBED + k * 16, 16)] = accs[k] * 0.125
        return _
    lax.fori_loop(0, BPW // 16, genre_body, None)

    for cp in tcopies:
        cp.wait()

    def merge_body(m, _):
        for b in range(16):
            for k in range(EMBED // 16):
                outbuf_v[m * 16 + b, pl.ds(k * 16, 16)] = (
                    trows_v[m * 16 + b, pl.ds(k * 16, 16)])
        return _
    lax.fori_loop(0, BPW // 16, merge_body, None)

    pltpu.sync_copy(outbuf_v, out_hbm.at[pl.ds(base, BPW)])


@jax.jit
def _run(title_table, genre_table, tidx, gidx):
    mesh = plsc.VectorSubcoreMesh(core_axis_name="c", subcore_axis_name="s",
                                  num_cores=NC, num_subcores=NS)
    return pl.kernel(
        _body,
        out_type=jax.ShapeDtypeStruct((B, 2 * EMBED), jnp.float32),
        mesh=mesh,
        scratch_types=[
            pltpu.VMEM((T_CHUNKS, IDX_CHUNK), jnp.int32),
            pltpu.VMEM((BPW, N_GENRES), jnp.int32),
            pltpu.VMEM((GENRE_ROWS, EMBED), jnp.float32),
            pltpu.VMEM((BPW, EMBED), jnp.float32),
            pltpu.VMEM((BPW, 2 * EMBED), jnp.float32),
            pltpu.SemaphoreType.DMA,
        ],
        compiler_params=pltpu.CompilerParams(use_tc_tiling_on_sc=False,
                                             needs_layout_passes=False),
    )(title_table, genre_table, tidx, gidx)


def kernel(title_table, genre_table, movie_title, movie_genres):
    return _run(title_table, genre_table,
                movie_title.astype(jnp.int32),
                movie_genres.astype(jnp.int32))
